# Initial kernel scaffold; baseline (speedup 1.0000x reference)
import jax
import jax.numpy as jnp
from jax.experimental import pallas as pl


def kernel(x, edge_index, batch, W1, a_src1, a_dst1, b1, g1, be1, W2, a_src2, a_dst2, b2, g2, be2, W3, a_src3, a_dst3, b3, g3, be3):
    raise NotImplementedError("write your pallas kernel here")


# speedup vs baseline: 2.8937x; 2.8937x over previous
import functools
import math

import jax
import jax.numpy as jnp
from jax.experimental import pallas as pl
from jax.experimental.pallas import tpu as pltpu

N = 10000
E = 160000
G = 128
HEADS = 4
F_IN = 9
HID = 192
OUT = 384
EPS_BN = 1e-5

_ROWS = 1000
_POOL_ROWS = 1000


def _dense_body(x_ref, w_ref, as_ref, ad_ref, h_ref, s_ref, d_ref):
    h = jnp.dot(x_ref[...], w_ref[...], preferred_element_type=jnp.float32)
    h_ref[...] = h
    s_ref[...] = jnp.dot(h, as_ref[...], preferred_element_type=jnp.float32)
    d_ref[...] = jnp.dot(h, ad_ref[...], preferred_element_type=jnp.float32)


def _dense(x, w, a_src_m, a_dst_m):
    n, fin = x.shape
    fout = w.shape[1]
    grid = n // _ROWS
    return pl.pallas_call(
        _dense_body,
        grid=(grid,),
        in_specs=[
            pl.BlockSpec((_ROWS, fin), lambda i: (i, 0)),
            pl.BlockSpec((fin, fout), lambda i: (0, 0)),
            pl.BlockSpec((fout, HEADS), lambda i: (0, 0)),
            pl.BlockSpec((fout, HEADS), lambda i: (0, 0)),
        ],
        out_specs=[
            pl.BlockSpec((_ROWS, fout), lambda i: (i, 0)),
            pl.BlockSpec((_ROWS, HEADS), lambda i: (i, 0)),
            pl.BlockSpec((_ROWS, HEADS), lambda i: (i, 0)),
        ],
        out_shape=[
            jax.ShapeDtypeStruct((n, fout), jnp.float32),
            jax.ShapeDtypeStruct((n, HEADS), jnp.float32),
            jax.ShapeDtypeStruct((n, HEADS), jnp.float32),
        ],
    )(x, w, a_src_m, a_dst_m)


def _epilogue_body(agg_ref, mm_ref, sc_ref, sh_ref, o_ref):
    m = jnp.dot(agg_ref[...], mm_ref[...], preferred_element_type=jnp.float32)
    o_ref[...] = jnp.maximum(m * sc_ref[...] + sh_ref[...], 0.0)


def _epilogue(agg, mmean, scale, shift):
    n, hf = agg.shape
    c = mmean.shape[1]
    return pl.pallas_call(
        _epilogue_body,
        grid=(n // _ROWS,),
        in_specs=[
            pl.BlockSpec((_ROWS, hf), lambda i: (i, 0)),
            pl.BlockSpec((hf, c), lambda i: (0, 0)),
            pl.BlockSpec((1, c), lambda i: (0, 0)),
            pl.BlockSpec((1, c), lambda i: (0, 0)),
        ],
        out_specs=pl.BlockSpec((_ROWS, c), lambda i: (i, 0)),
        out_shape=jax.ShapeDtypeStruct((n, c), jnp.float32),
    )(agg, mmean, scale, shift)


def _pool_body(b_ref, h_ref, o_ref, acc_ref, cnt_ref):
    i = pl.program_id(0)

    @pl.when(i == 0)
    def _init():
        acc_ref[...] = jnp.zeros_like(acc_ref)
        cnt_ref[...] = jnp.zeros_like(cnt_ref)

    b = b_ref[0]
    gids = jax.lax.broadcasted_iota(jnp.int32, (G, 1), 0)
    mask = (gids == b).astype(jnp.float32)
    acc_ref[...] += jnp.dot(mask, h_ref[...], preferred_element_type=jnp.float32)
    cnt_ref[...] += jnp.sum(mask, axis=1, keepdims=True)

    @pl.when(i == pl.num_programs(0) - 1)
    def _fin():
        o_ref[...] = acc_ref[...] / jnp.maximum(cnt_ref[...], 1.0)


def _pool(batch2d, h):
    n = h.shape[0] // 1
    c = h.shape[1]
    nblk = batch2d.shape[0]
    return pl.pallas_call(
        _pool_body,
        grid=(nblk,),
        in_specs=[
            pl.BlockSpec((1, 1, _POOL_ROWS), lambda i: (i, 0, 0)),
            pl.BlockSpec((_POOL_ROWS, c), lambda i: (i, 0)),
        ],
        out_specs=pl.BlockSpec((G, c), lambda i: (0, 0)),
        out_shape=jax.ShapeDtypeStruct((G, c), jnp.float32),
        scratch_shapes=[
            pltpu.VMEM((G, c), jnp.float32),
            pltpu.VMEM((G, 1), jnp.float32),
        ],
    )(batch2d, h)


def _edge_phase_xla(hmat, a_s, a_d, src, dst, heads, ch):
    e = a_s[src] + a_d[dst]
    e = jnp.where(e > 0, e, 0.2 * e)
    p = jnp.exp(e)
    denom = jax.ops.segment_sum(p, dst, num_segments=N)
    alpha = p / (denom[dst] + 1e-16)
    msg = hmat[src].reshape(-1, heads, ch) * alpha[:, :, None]
    agg = jax.ops.segment_sum(msg.reshape(-1, heads * ch), dst, num_segments=N)
    return agg


def _mk_alpha_mat(a):
    heads, ch = a.shape
    eye = jnp.eye(heads, dtype=a.dtype)
    return (a[:, :, None] * eye[:, None, :]).reshape(heads * ch, heads)


def _mk_mean_mat(heads, ch, dtype=jnp.float32):
    eye = jnp.eye(ch, dtype=dtype) / heads
    return jnp.tile(eye, (heads, 1))


def kernel(x, edge_index, batch,
           W1, a_src1, a_dst1, b1, g1, be1,
           W2, a_src2, a_dst2, b2, g2, be2,
           W3, a_src3, a_dst3, b3, g3, be3):
    loop = jnp.arange(N, dtype=edge_index.dtype)
    src = jnp.concatenate([edge_index[0], loop])
    dst = jnp.concatenate([edge_index[1], loop])

    k = 1.0 / math.sqrt(1.0 + EPS_BN)
    h = x
    params = [
        (W1, a_src1, a_dst1, b1, g1, be1, HID),
        (W2, a_src2, a_dst2, b2, g2, be2, HID),
        (W3, a_src3, a_dst3, b3, g3, be3, OUT),
    ]
    for (W, a_s, a_d, b, g, be, ch) in params:
        hm, al_s, al_d = _dense(h, W, _mk_alpha_mat(a_s), _mk_alpha_mat(a_d))
        agg = _edge_phase_xla(hm, al_s, al_d, src, dst, HEADS, ch)
        scale = (k * g).reshape(1, ch)
        shift = (be + k * g * b).reshape(1, ch)
        h = _epilogue(agg, _mk_mean_mat(HEADS, ch), scale, shift)

    return _pool(batch.reshape(-1, 1, _POOL_ROWS), h)


# speedup vs baseline: 8.5972x; 2.9710x over previous
import functools
import math

import jax
import jax.numpy as jnp
from jax import lax
from jax.experimental import pallas as pl
from jax.experimental.pallas import tpu as pltpu
from jax.experimental.pallas import tpu_sc as plsc

N = 10000
E = 160000
G = 128
HEADS = 4
F_IN = 9
HID = 192
OUT = 384
EPS_BN = 1e-5

_ROWS = 1000
_POOL_ROWS = 1000
_NW = 32
_NPW = 320
_K = 64
_E2 = E + N
_EPAD = _E2 + 2 * _K


def _dense_body(fout, x_ref, w_ref, as_ref, ad_ref, h_ref, d_ref):
    h = jnp.dot(x_ref[...], w_ref[...], preferred_element_type=jnp.float32)
    h_ref[:, pl.ds(0, fout)] = h
    h_ref[:, pl.ds(fout, 128)] = jnp.dot(
        h, as_ref[...], preferred_element_type=jnp.float32)
    d_ref[...] = jnp.dot(h, ad_ref[...], preferred_element_type=jnp.float32)


def _dense(x, w, a_src_m, a_dst_m):
    n, fin = x.shape
    fout = w.shape[1]
    return pl.pallas_call(
        functools.partial(_dense_body, fout),
        grid=(n // _ROWS,),
        in_specs=[
            pl.BlockSpec((_ROWS, fin), lambda i: (i, 0)),
            pl.BlockSpec((fin, fout), lambda i: (0, 0)),
            pl.BlockSpec((fout, 128), lambda i: (0, 0)),
            pl.BlockSpec((fout, 16), lambda i: (0, 0)),
        ],
        out_specs=[
            pl.BlockSpec((_ROWS, fout + 128), lambda i: (i, 0)),
            pl.BlockSpec((_ROWS, 16), lambda i: (i, 0)),
        ],
        out_shape=[
            jax.ShapeDtypeStruct((n, fout + 128), jnp.float32),
            jax.ShapeDtypeStruct((n, 16), jnp.float32),
        ],
    )(x, w, a_src_m, a_dst_m)


def _pool_body(b_ref, h_ref, o_ref, acc_ref, cnt_ref):
    i = pl.program_id(0)

    @pl.when(i == 0)
    def _init():
        acc_ref[...] = jnp.zeros_like(acc_ref)
        cnt_ref[...] = jnp.zeros_like(cnt_ref)

    b = b_ref[0]
    gids = lax.broadcasted_iota(jnp.int32, (G, 1), 0)
    mask = (gids == b).astype(jnp.float32)
    acc_ref[...] += jnp.dot(mask, h_ref[...], preferred_element_type=jnp.float32)
    cnt_ref[...] += jnp.sum(mask, axis=1, keepdims=True)

    @pl.when(i == pl.num_programs(0) - 1)
    def _fin():
        o_ref[...] = acc_ref[...] / jnp.maximum(cnt_ref[...], 1.0)


def _pool(batch3d, h):
    c = h.shape[1]
    nblk = batch3d.shape[0]
    return pl.pallas_call(
        _pool_body,
        grid=(nblk,),
        in_specs=[
            pl.BlockSpec((1, 1, _POOL_ROWS), lambda i: (i, 0, 0)),
            pl.BlockSpec((_POOL_ROWS, c), lambda i: (i, 0)),
        ],
        out_specs=pl.BlockSpec((G, c), lambda i: (0, 0)),
        out_shape=jax.ShapeDtypeStruct((G, c), jnp.float32),
        scratch_shapes=[
            pltpu.VMEM((G, c), jnp.float32),
            pltpu.VMEM((G, 1), jnp.float32),
        ],
    )(batch3d, h)


@functools.lru_cache(maxsize=None)
def _sc_edge_builder(c, kch):
    return _sc_edge_build(c, kch)


def _sc_edge(hmc, ald, src_s, dst_s, bounds, scale, shift, c, kch):
    return _sc_edge_builder(c, kch)(
        hmc, ald, src_s, dst_s, bounds, scale, shift)


def _sc_edge_build(c, kch):
    hf = HEADS * c
    hw = hf + 128
    nsl = hf // 16
    csl = c // 16
    mesh = plsc.VectorSubcoreMesh(core_axis_name="c", subcore_axis_name="s")

    @functools.partial(
        pl.kernel, mesh=mesh,
        out_type=jax.ShapeDtypeStruct((N, c), jnp.float32),
        scratch_types=[
            pltpu.SMEM((48,), jnp.int32),
            pltpu.VMEM((48,), jnp.int32),
            pltpu.VMEM((kch,), jnp.int32),
            pltpu.VMEM((kch,), jnp.int32),
            pltpu.SMEM((kch,), jnp.int32),
            pltpu.VMEM((_NPW, 16), jnp.float32),
            pltpu.VMEM((kch, hw), jnp.float32),
            pltpu.VMEM((hf,), jnp.float32),
            pltpu.VMEM((16,), jnp.float32),
            pltpu.VMEM((8, c), jnp.float32),
            pltpu.VMEM((c,), jnp.float32),
            pltpu.VMEM((c,), jnp.float32),
            pltpu.SemaphoreType.DMA,
        ],
    )
    def edge_kernel(hm_r, ald_r, srcs_r, dsts_r, bnd_r, scl_r, shf_r,
                    out_r, bnd_s, bnd_v, src_v, dst_v, dst_s, ald_loc,
                    h_v, acc_v, dacc_v, stage_v, scl_v, shf_v, sem):
        wid = lax.axis_index("s") * 2 + lax.axis_index("c")
        n0 = wid * _NPW
        pltpu.sync_copy(bnd_r, bnd_v)
        for g in range(48 // 16):
            grp = bnd_v[pl.ds(16 * g, 16)]
            for ln in range(16):
                bnd_s[16 * g + ln] = grp[ln]
        pltpu.sync_copy(scl_r, scl_v)
        pltpu.sync_copy(shf_r, shf_v)
        pltpu.sync_copy(ald_r.at[pl.ds(n0, _NPW)], ald_loc)
        e0 = bnd_s[wid]
        e1 = bnd_s[wid + 1]
        base0 = (e0 // 8) * 8
        nch = (e1 - base0 + kch - 1) // kch
        zero16 = jnp.zeros((16,), jnp.float32)
        for t in range(nsl):
            acc_v[pl.ds(16 * t, 16)] = zero16
        dacc_v[...] = zero16

        def _flush(node):
            r = node % 8
            rv = 0.25 / (dacc_v[...] + 1e-16)
            for t in range(csl):
                sl = pl.ds(16 * t, 16)
                o = acc_v[pl.ds(16 * t, 16)] * rv[0]
                for h2 in range(1, HEADS):
                    o = o + acc_v[pl.ds(h2 * c + 16 * t, 16)] * rv[h2]
                stage_v[r, sl] = jnp.maximum(o * scl_v[sl] + shf_v[sl], 0.0)
            for t in range(nsl):
                acc_v[pl.ds(16 * t, 16)] = zero16
            dacc_v[...] = zero16

            @pl.when(r == 7)
            def _write_group():
                g0 = pl.multiple_of(node - 7, 8)
                pltpu.sync_copy(stage_v, out_r.at[pl.ds(g0, 8)])

        def _chunk(j, cur):
            base = pl.multiple_of(base0 + j * kch, 8)
            pltpu.sync_copy(srcs_r.at[pl.ds(base, kch)], src_v)
            pltpu.sync_copy(dsts_r.at[pl.ds(base, kch)], dst_v)
            for g in range(kch // 16):
                grp = dst_v[pl.ds(16 * g, 16)]
                for ln in range(16):
                    dst_s[16 * g + ln] = grp[ln]
            pltpu.async_copy(hm_r.at[src_v], h_v, sem).wait()

            def _edge(il, cur):
                dnew = dst_s[il]

                def _do_flush():
                    _flush(cur)
                    return dnew

                cur = lax.cond(dnew != cur, _do_flush, lambda: cur)
                ev = h_v[il, pl.ds(hf, 16)] + ald_loc[dnew - n0]
                ev = jnp.maximum(ev, 0.2 * ev)
                pv = jnp.exp(ev)
                dacc_v[...] += pv
                for h2 in range(HEADS):
                    ph = pv[h2]
                    for t in range(csl):
                        sl = pl.ds(h2 * c + 16 * t, 16)
                        acc_v[sl] += ph * h_v[il, sl]
                return cur

            lo = jnp.maximum(e0 - base, 0)
            hi = jnp.minimum(e1 - base, kch)
            return lax.fori_loop(lo, hi, _edge, cur)

        cur = lax.fori_loop(0, nch, _chunk, n0)
        _flush(cur)

    return edge_kernel


def _mk_alpha_mat(a, width):
    heads, ch = a.shape
    eye = jnp.concatenate(
        [jnp.eye(heads, dtype=a.dtype),
         jnp.zeros((heads, width - heads), a.dtype)], axis=1)
    return (a[:, :, None] * eye[:, None, :]).reshape(heads * ch, width)


def kernel(x, edge_index, batch,
           W1, a_src1, a_dst1, b1, g1, be1,
           W2, a_src2, a_dst2, b2, g2, be2,
           W3, a_src3, a_dst3, b3, g3, be3):
    loop = jnp.arange(N, dtype=edge_index.dtype)
    src = jnp.concatenate([edge_index[0], loop])
    dst = jnp.concatenate([edge_index[1], loop])
    perm = jnp.argsort(dst)
    ssrc = src[perm]
    sdst = dst[perm]
    marks = jnp.clip(jnp.arange(33) * _NPW, 0, N)
    bounds = jnp.searchsorted(sdst, marks).astype(jnp.int32)
    bounds = jnp.concatenate([bounds, jnp.full((15,), _E2, jnp.int32)])
    pad = jnp.zeros((_EPAD - _E2,), jnp.int32)
    ssrc_p = jnp.concatenate([ssrc, pad])
    sdst_p = jnp.concatenate([sdst, pad])

    k = 1.0 / math.sqrt(1.0 + EPS_BN)
    h = x
    params = [
        (W1, a_src1, a_dst1, b1, g1, be1, HID),
        (W2, a_src2, a_dst2, b2, g2, be2, HID),
        (W3, a_src3, a_dst3, b3, g3, be3, OUT),
    ]
    for (W, a_s, a_d, b, g, be, ch) in params:
        hmc, al_d = _dense(h, W, _mk_alpha_mat(a_s, 128), _mk_alpha_mat(a_d, 16))
        al_d = jnp.concatenate(
            [al_d, jnp.zeros((_NW * _NPW - N, 16), jnp.float32)])
        scale = k * g
        shift = be + k * g * b
        kch = 64 if ch <= HID else 48
        h = _sc_edge(hmc, al_d, ssrc_p, sdst_p, bounds, scale, shift, ch, kch)

    return _pool(batch.reshape(-1, 1, _POOL_ROWS), h)


# speedup vs baseline: 9.5203x; 1.1074x over previous
import functools
import math

import jax
import jax.numpy as jnp
from jax import lax
from jax.experimental import pallas as pl
from jax.experimental.pallas import tpu as pltpu
from jax.experimental.pallas import tpu_sc as plsc

N = 10000
E = 160000
G = 128
HEADS = 4
F_IN = 9
HID = 192
OUT = 384
EPS_BN = 1e-5

_ROWS = 1000
_POOL_ROWS = 1000
_NW = 32
_NPW = 320
_K = 64
_E2 = E + N
_EPAD = _E2 + 2 * _K


def _dense_body(fout, x_ref, w_ref, as_ref, ad_ref, h_ref, s_ref, d_ref):
    h = jnp.dot(x_ref[...], w_ref[...], preferred_element_type=jnp.float32)
    h_ref[:, pl.ds(0, fout)] = h
    als = jnp.dot(h, as_ref[...], preferred_element_type=jnp.float32)
    h_ref[:, pl.ds(fout, 128)] = als
    s_ref[...] = als
    d_ref[...] = jnp.dot(h, ad_ref[...], preferred_element_type=jnp.float32)


def _dense(x, w, a_src_m, a_dst_m):
    n, fin = x.shape
    fout = w.shape[1]
    return pl.pallas_call(
        functools.partial(_dense_body, fout),
        grid=(n // _ROWS,),
        in_specs=[
            pl.BlockSpec((_ROWS, fin), lambda i: (i, 0)),
            pl.BlockSpec((fin, fout), lambda i: (0, 0)),
            pl.BlockSpec((fout, 128), lambda i: (0, 0)),
            pl.BlockSpec((fout, 16), lambda i: (0, 0)),
        ],
        out_specs=[
            pl.BlockSpec((_ROWS, fout + 128), lambda i: (i, 0)),
            pl.BlockSpec((_ROWS, 128), lambda i: (i, 0)),
            pl.BlockSpec((_ROWS, 16), lambda i: (i, 0)),
        ],
        out_shape=[
            jax.ShapeDtypeStruct((n, fout + 128), jnp.float32),
            jax.ShapeDtypeStruct((n, 128), jnp.float32),
            jax.ShapeDtypeStruct((n, 16), jnp.float32),
        ],
    )(x, w, a_src_m, a_dst_m)


def _pool_body(b_ref, h_ref, o_ref, acc_ref, cnt_ref):
    i = pl.program_id(0)

    @pl.when(i == 0)
    def _init():
        acc_ref[...] = jnp.zeros_like(acc_ref)
        cnt_ref[...] = jnp.zeros_like(cnt_ref)

    b = b_ref[0]
    gids = lax.broadcasted_iota(jnp.int32, (G, 1), 0)
    mask = (gids == b).astype(jnp.float32)
    acc_ref[...] += jnp.dot(mask, h_ref[...], preferred_element_type=jnp.float32)
    cnt_ref[...] += jnp.sum(mask, axis=1, keepdims=True)

    @pl.when(i == pl.num_programs(0) - 1)
    def _fin():
        o_ref[...] = acc_ref[...] / jnp.maximum(cnt_ref[...], 1.0)


def _pool(batch3d, h):
    c = h.shape[1]
    nblk = batch3d.shape[0]
    return pl.pallas_call(
        _pool_body,
        grid=(nblk,),
        in_specs=[
            pl.BlockSpec((1, 1, _POOL_ROWS), lambda i: (i, 0, 0)),
            pl.BlockSpec((_POOL_ROWS, c), lambda i: (i, 0)),
        ],
        out_specs=pl.BlockSpec((G, c), lambda i: (0, 0)),
        out_shape=jax.ShapeDtypeStruct((G, c), jnp.float32),
        scratch_shapes=[
            pltpu.VMEM((G, c), jnp.float32),
            pltpu.VMEM((G, 1), jnp.float32),
        ],
    )(batch3d, h)


@functools.lru_cache(maxsize=None)
def _sc_edge_builder(c, kch, nsub):
    hf = HEADS * c
    hw = hf + 128
    csl = c // 16
    ns = _NPW // nsub
    mesh = plsc.VectorSubcoreMesh(core_axis_name="c", subcore_axis_name="s")

    @functools.partial(
        pl.kernel, mesh=mesh,
        out_type=jax.ShapeDtypeStruct((_NW * _NPW * c,), jnp.float32),
        scratch_types=[
            pltpu.SMEM((80,), jnp.int32),
            pltpu.VMEM((80,), jnp.int32),
            pltpu.VMEM((kch,), jnp.int32),
            pltpu.VMEM((kch,), jnp.int32),
            pltpu.SMEM((kch,), jnp.int32),
            pltpu.VMEM((_NPW * 16,), jnp.float32),
            pltpu.VMEM((kch, 128), jnp.float32),
            pltpu.VMEM((kch, hw), jnp.float32),
            pltpu.VMEM((nsub * c,), jnp.float32),
            pltpu.VMEM((nsub * 16,), jnp.float32),
            pltpu.VMEM((c,), jnp.float32),
            pltpu.VMEM((c,), jnp.float32),
            pltpu.SemaphoreType.DMA,
        ],
    )
    def edge_kernel(hm_r, als_r, ald_r, srcs_r, dsts_r, bnd_r, scl_r, shf_r,
                    out_r, bnd_s, bnd_v, src_v, dst_v, dst_s, ald_loc,
                    as_v, h_v, acc_v, dacc_v, scl_v, shf_v, sem):
        wid = lax.axis_index("s") * 2 + lax.axis_index("c")
        n0 = wid * _NPW
        pltpu.sync_copy(bnd_r, bnd_v)

        def _bgrp(g, _):
            grp = bnd_v[pl.ds(16 * g, 16)]
            for ln in range(16):
                bnd_s[16 * g + ln] = grp[ln]
            return 0

        lax.fori_loop(0, 80 // 16, _bgrp, 0)
        pltpu.sync_copy(scl_r, scl_v)
        pltpu.sync_copy(shf_r, shf_v)
        pltpu.sync_copy(ald_r.at[pl.ds(n0 * 16, _NPW * 16)], ald_loc)
        zero16 = jnp.zeros((16,), jnp.float32)

        def _load_chunk(base):
            pltpu.sync_copy(srcs_r.at[pl.ds(base, kch)], src_v)
            pltpu.sync_copy(dsts_r.at[pl.ds(base, kch)], dst_v)
            for g in range(kch // 16):
                grp = dst_v[pl.ds(16 * g, 16)]
                for ln in range(16):
                    dst_s[16 * g + ln] = grp[ln]

        def _subrange(s, _):
            sub = wid * ns + s
            nsub0 = n0 + s * nsub
            e0 = bnd_s[sub]
            e1 = bnd_s[sub + 1]
            base0 = (e0 // 8) * 8
            nch = (e1 - base0 + kch - 1) // kch

            def _zrow(r, _):
                for t in range(csl):
                    acc_v[pl.ds(r * c + 16 * t, 16)] = zero16
                dacc_v[pl.ds(r * 16, 16)] = zero16
                return 0

            lax.fori_loop(0, nsub, _zrow, 0)

            def _p1chunk(j, _):
                base = pl.multiple_of(base0 + j * kch, 8)
                _load_chunk(base)
                pltpu.async_copy(als_r.at[src_v], as_v, sem).wait()

                def _edge(il, _):
                    d = dst_s[il]
                    ev = as_v[il, pl.ds(0, 16)] + ald_loc[pl.ds((d - n0) * 16, 16)]
                    ev = jnp.maximum(ev, 0.2 * ev)
                    dacc_v[pl.ds((d - nsub0) * 16, 16)] += jnp.exp(ev)
                    return 0

                lo = jnp.maximum(e0 - base, 0)
                hi = jnp.minimum(e1 - base, kch)
                return lax.fori_loop(lo, hi, _edge, 0)

            lax.fori_loop(0, nch, _p1chunk, 0)

            def _inv(r, _):
                sl = pl.ds(r * 16, 16)
                dacc_v[sl] = 0.25 / (dacc_v[sl] + 1e-16)
                return 0

            lax.fori_loop(0, nsub, _inv, 0)

            def _p2chunk(j, _):
                base = pl.multiple_of(base0 + j * kch, 8)
                _load_chunk(base)
                pltpu.async_copy(hm_r.at[src_v], h_v, sem).wait()

                def _edge(il, _):
                    d = dst_s[il]
                    dl = d - nsub0
                    ev = h_v[il, pl.ds(hf, 16)] + ald_loc[pl.ds((d - n0) * 16, 16)]
                    ev = jnp.maximum(ev, 0.2 * ev)
                    av = jnp.exp(ev) * dacc_v[pl.ds(dl * 16, 16)]
                    for t in range(csl):
                        m = av[0] * h_v[il, pl.ds(16 * t, 16)]
                        for h2 in range(1, HEADS):
                            m = m + av[h2] * h_v[il, pl.ds(h2 * c + 16 * t, 16)]
                        acc_v[pl.ds(dl * c + 16 * t, 16)] += m
                    return 0

                lo = jnp.maximum(e0 - base, 0)
                hi = jnp.minimum(e1 - base, kch)
                return lax.fori_loop(lo, hi, _edge, 0)

            lax.fori_loop(0, nch, _p2chunk, 0)

            def _epi(r, _):
                for t in range(csl):
                    sl = pl.ds(r * c + 16 * t, 16)
                    cs = pl.ds(16 * t, 16)
                    acc_v[sl] = jnp.maximum(
                        acc_v[sl] * scl_v[cs] + shf_v[cs], 0.0)
                return 0

            lax.fori_loop(0, nsub, _epi, 0)
            pltpu.sync_copy(
                acc_v,
                out_r.at[pl.ds(pl.multiple_of(nsub0 * c, 8), nsub * c)])
            return 0

        lax.fori_loop(0, ns, _subrange, 0)

    return edge_kernel


def _mk_alpha_mat(a, width):
    heads, ch = a.shape
    eye = jnp.concatenate(
        [jnp.eye(heads, dtype=a.dtype),
         jnp.zeros((heads, width - heads), a.dtype)], axis=1)
    return (a[:, :, None] * eye[:, None, :]).reshape(heads * ch, width)


def kernel(x, edge_index, batch,
           W1, a_src1, a_dst1, b1, g1, be1,
           W2, a_src2, a_dst2, b2, g2, be2,
           W3, a_src3, a_dst3, b3, g3, be3):
    loop = jnp.arange(N, dtype=edge_index.dtype)
    src = jnp.concatenate([edge_index[0], loop])
    dst = jnp.concatenate([edge_index[1], loop])
    perm = jnp.argsort(dst)
    ssrc = src[perm]
    sdst = dst[perm]
    def _mk_bounds(nsub):
        nm = (_NW * _NPW) // nsub
        marks = jnp.clip(jnp.arange(nm + 1) * nsub, 0, N)
        b = jnp.searchsorted(sdst, marks).astype(jnp.int32)
        return jnp.concatenate([b, jnp.full((80 - nm - 1,), _E2, jnp.int32)])

    bounds_by_nsub = {320: _mk_bounds(320), 160: _mk_bounds(160)}
    pad = jnp.zeros((_EPAD - _E2,), jnp.int32)
    ssrc_p = jnp.concatenate([ssrc, pad])
    sdst_p = jnp.concatenate([sdst, pad])

    k = 1.0 / math.sqrt(1.0 + EPS_BN)
    h = x
    params = [
        (W1, a_src1, a_dst1, b1, g1, be1, HID),
        (W2, a_src2, a_dst2, b2, g2, be2, HID),
        (W3, a_src3, a_dst3, b3, g3, be3, OUT),
    ]
    for (W, a_s, a_d, b, g, be, ch) in params:
        hmc, al_s, al_d = _dense(
            h, W, _mk_alpha_mat(a_s, 128), _mk_alpha_mat(a_d, 16))
        al_d = jnp.concatenate(
            [al_d, jnp.zeros((_NW * _NPW - N, 16), jnp.float32)])
        scale = k * g
        shift = be + k * g * b
        kch, nsub = (16, 320) if ch <= HID else (16, 160)
        h = _sc_edge_builder(ch, kch, nsub)(
            hmc, al_s, al_d.reshape(-1), ssrc_p, sdst_p, bounds_by_nsub[nsub],
            scale, shift).reshape(_NW * _NPW, ch)[:N]

    return _pool(batch.reshape(-1, 1, _POOL_ROWS), h)
